# baseline (speedup 1.0000x reference)
import jax
import jax.numpy as jnp
from jax.experimental import pallas as pl
from jax.experimental.pallas import tpu as pltpu
from jax.experimental.pallas import tpu_sc as plsc

_N_ROWS = 16384
_N_COLS = 2048
_N_OUT = (_N_ROWS // 2) * _N_COLS

_BLK_E = 16384
_N_BLOCKS = _N_OUT // _BLK_E
_LANES = 16
_UNROLL = 16


def _sc_mean(x_view):
    mesh = plsc.VectorSubcoreMesh(core_axis_name="core", subcore_axis_name="subcore")

    @pl.kernel(
        out_type=jax.ShapeDtypeStruct((_N_BLOCKS, _BLK_E), jnp.float32),
        mesh=mesh,
    )
    def run(x_hbm, y_hbm, o_hbm):
        def body(x1_v, x2_v, o_v):
            @pl.loop(0, _BLK_E, step=_LANES * _UNROLL)
            def _(c0):
                for u in range(_UNROLL):
                    s = pl.ds(c0 + u * _LANES, _LANES)
                    o_v[0, s] = (x1_v[0, s] + x2_v[0, s]) * 0.5

        pltpu.emit_pipeline(
            body,
            grid=(_N_BLOCKS,),
            in_specs=[
                pl.BlockSpec((1, _BLK_E), lambda i: (i, 0)),
                pl.BlockSpec((1, _BLK_E), lambda i: (i + _N_BLOCKS, 0)),
            ],
            out_specs=[pl.BlockSpec((1, _BLK_E), lambda i: (i, 0))],
            core_axis_name=("core", "subcore"),
            dimension_semantics=(pltpu.PARALLEL,),
        )(x_hbm, y_hbm, o_hbm)

    return run(x_view, x_view)


def kernel(x_cat):
    x_view = x_cat.reshape(2 * _N_BLOCKS, _BLK_E)
    out = _sc_mean(x_view)
    return out.reshape(_N_ROWS // 2, _N_COLS)


# speedup vs baseline: 1.1523x; 1.1523x over previous
import jax
import jax.numpy as jnp
from jax.experimental import pallas as pl
from jax.experimental.pallas import tpu as pltpu
from jax.experimental.pallas import tpu_sc as plsc

_N_ROWS = 16384
_N_COLS = 2048
_N_OUT = (_N_ROWS // 2) * _N_COLS

_BLK_E = 16384
_N_BLOCKS = _N_OUT // _BLK_E
_LANES = 16
_UNROLL = 16


def _sc_mean(x_flat):
    mesh = plsc.VectorSubcoreMesh(core_axis_name="core", subcore_axis_name="subcore")

    @pl.kernel(
        out_type=jax.ShapeDtypeStruct((_N_OUT,), jnp.float32),
        mesh=mesh,
        compiler_params=pltpu.CompilerParams(needs_layout_passes=False),
    )
    def run(x_hbm, y_hbm, o_hbm):
        def body(x1_v, x2_v, o_v):
            @pl.loop(0, _BLK_E, step=_LANES * _UNROLL)
            def _(c0):
                for u in range(_UNROLL):
                    s = pl.ds(c0 + u * _LANES, _LANES)
                    o_v[s] = (x1_v[s] + x2_v[s]) * 0.5

        pltpu.emit_pipeline(
            body,
            grid=(_N_BLOCKS,),
            in_specs=[
                pl.BlockSpec((_BLK_E,), lambda i: (i,)),
                pl.BlockSpec((_BLK_E,), lambda i: (i + _N_BLOCKS,)),
            ],
            out_specs=[pl.BlockSpec((_BLK_E,), lambda i: (i,))],
            core_axis_name=("core", "subcore"),
            dimension_semantics=(pltpu.PARALLEL,),
        )(x_hbm, y_hbm, o_hbm)

    return run(x_flat, x_flat)


def kernel(x_cat):
    x_flat = x_cat.reshape(-1)
    out = _sc_mean(x_flat)
    return out.reshape(_N_ROWS // 2, _N_COLS)


# speedup vs baseline: 5.1770x; 4.4927x over previous
import jax
import jax.numpy as jnp
from jax.experimental import pallas as pl
from jax.experimental.pallas import tpu as pltpu
from jax.experimental.pallas import tpu_sc as plsc

_N_ROWS = 16384
_N_COLS = 2048
_OUT_ROWS = _N_ROWS // 2

_BLK_R = 8
_N_BLOCKS = _OUT_ROWS // _BLK_R
_LANES = 16


def _sc_mean(x_cat):
    mesh = plsc.VectorSubcoreMesh(core_axis_name="core", subcore_axis_name="subcore")

    @pl.kernel(
        out_type=jax.ShapeDtypeStruct((_OUT_ROWS, _N_COLS), jnp.float32),
        mesh=mesh,
        compiler_params=pltpu.CompilerParams(needs_layout_passes=False),
    )
    def run(x_hbm, y_hbm, o_hbm):
        def body(x1_v, x2_v, o_v):
            for r in range(_BLK_R):
                @plsc.parallel_loop(0, _N_COLS, _LANES, unroll=8)
                def _(c):
                    s = pl.ds(c, _LANES)
                    o_v[r, s] = (x1_v[r, s] + x2_v[r, s]) * 0.5

        pltpu.emit_pipeline(
            body,
            grid=(_N_BLOCKS,),
            in_specs=[
                pl.BlockSpec((_BLK_R, _N_COLS), lambda i: (i, 0)),
                pl.BlockSpec((_BLK_R, _N_COLS), lambda i: (i + _N_BLOCKS, 0)),
            ],
            out_specs=[pl.BlockSpec((_BLK_R, _N_COLS), lambda i: (i, 0))],
            core_axis_name=("core", "subcore"),
            dimension_semantics=(pltpu.PARALLEL,),
        )(x_hbm, y_hbm, o_hbm)

    return run(x_cat, x_cat)


def kernel(x_cat):
    return _sc_mean(x_cat)
